# Initial kernel scaffold; baseline (speedup 1.0000x reference)
import jax
import jax.numpy as jnp
from jax.experimental import pallas as pl


def kernel(atomic_number, other_feats, edge_index, edge_attr, params):
    raise NotImplementedError("write your pallas kernel here")


# speedup vs baseline: 3.8479x; 3.8479x over previous
import functools

import jax
import jax.numpy as jnp
from jax import lax
from jax.experimental import pallas as pl
from jax.experimental.pallas import tpu as pltpu
from jax.experimental.pallas import tpu_sc as plsc

N = 10000
E = 320000
NUM_ATOM_TYPES = 100
EMB = 64
HID = 256
EDGE_DIM = 16
D0 = 128

NCORE = 2
NSUB = 16
CHUNK = 40
EPT = E // NSUB
NCH = EPT // CHUNK
RPT = N // NSUB
ZROWS = 25

NB = 2000
EB = 2000


def _embed_body(ids_ref, feats_ref, emb_ref, x_ref):
    ids = ids_ref[...]
    iota = lax.broadcasted_iota(jnp.int32, (1, 104), 1)
    oh = (ids == iota).astype(jnp.float32)
    x = jnp.dot(oh, emb_ref[...], preferred_element_type=jnp.float32)
    x_ref[...] = x + feats_ref[...]


def _edge_body(ea_ref, w_ref, b_ref, e0_ref, e1_ref, e2_ref, e3_ref):
    e = jnp.dot(ea_ref[...], w_ref[...], preferred_element_type=jnp.float32)
    e = e + b_ref[...]
    e0_ref[...] = e[:, 0:128]
    e1_ref[0] = e[:, 128:256]
    e1_ref[1] = e[:, 256:384]
    e2_ref[0] = e[:, 384:512]
    e2_ref[1] = e[:, 512:640]
    e3_ref[0] = e[:, 640:768]
    e3_ref[1] = e[:, 768:896]


def _node_body(x_ref, agg_ref, w1_ref, b1_ref, w2_ref, b2_ref, h2_ref, st_ref):
    parts = [agg_ref[i] for i in range(agg_ref.shape[0])]
    aggc = parts[0] if len(parts) == 1 else jnp.concatenate(parts, axis=1)
    h = x_ref[...] + aggc
    h1 = jnp.dot(h, w1_ref[...], preferred_element_type=jnp.float32)
    h1 = jnp.maximum(h1 + b1_ref[...], 0.0)
    h2 = jnp.dot(h1, w2_ref[...], preferred_element_type=jnp.float32)
    h2 = h2 + b2_ref[...]
    h2_ref[...] = h2

    @pl.when(pl.program_id(0) == 0)
    def _():
        st_ref[...] = jnp.zeros_like(st_ref)

    s1 = jnp.sum(h2, axis=0, keepdims=True)
    s2 = jnp.sum(h2 * h2, axis=0, keepdims=True)
    st_ref[...] += jnp.concatenate(
        [s1, s2, jnp.zeros((6, HID), jnp.float32)], axis=0)


def _bn_body(h2_ref, st_ref, g_ref, b_ref, x_ref, xpk_ref, sum_ref):
    st = st_ref[...]
    mean = st[0:1, :] / N
    var = st[1:2, :] / N - mean * mean
    scale = g_ref[...] * lax.rsqrt(var + 1e-5)
    shift = b_ref[...] - mean * scale
    xn = jnp.maximum(h2_ref[...] * scale + shift, 0.0)
    x_ref[...] = xn
    xpk_ref[0] = xn[:, :128]
    xpk_ref[1] = xn[:, 128:]

    @pl.when(pl.program_id(0) == 0)
    def _():
        sum_ref[...] = jnp.zeros_like(sum_ref)

    sum_ref[...] += jnp.concatenate(
        [jnp.sum(xn, axis=0, keepdims=True), jnp.zeros((7, HID), jnp.float32)],
        axis=0)


def _head_body(sum_ref, w1_ref, b1_ref, w2_ref, b2_ref, o_ref):
    g = sum_ref[0:1, :] / N
    h = jnp.dot(g, w1_ref[...], preferred_element_type=jnp.float32)
    h = jnp.maximum(h + b1_ref[...], 0.0)
    o = jnp.dot(h, w2_ref[...], preferred_element_type=jnp.float32)
    o_ref[...] = o + b2_ref[...]


def _embed_call(ids, feats_pad, emb_wide):
    return pl.pallas_call(
        _embed_body,
        grid=(N // NB,),
        in_specs=[
            pl.BlockSpec((NB, 1), lambda i: (i, 0)),
            pl.BlockSpec((NB, D0), lambda i: (i, 0)),
            pl.BlockSpec((104, D0), lambda i: (0, 0)),
        ],
        out_specs=pl.BlockSpec((NB, D0), lambda i: (i, 0)),
        out_shape=jax.ShapeDtypeStruct((N, D0), jnp.float32),
    )(ids, feats_pad, emb_wide)


def _edge_call(edge_attr, w_all, b_all):
    return pl.pallas_call(
        _edge_body,
        grid=(E // EB,),
        in_specs=[
            pl.BlockSpec((EB, EDGE_DIM), lambda i: (i, 0)),
            pl.BlockSpec((EDGE_DIM, 896), lambda i: (0, 0)),
            pl.BlockSpec((1, 896), lambda i: (0, 0)),
        ],
        out_specs=[
            pl.BlockSpec((EB, 128), lambda i: (i, 0)),
            pl.BlockSpec((2, EB, 128), lambda i: (0, i, 0)),
            pl.BlockSpec((2, EB, 128), lambda i: (0, i, 0)),
            pl.BlockSpec((2, EB, 128), lambda i: (0, i, 0)),
        ],
        out_shape=[
            jax.ShapeDtypeStruct((E, 128), jnp.float32),
            jax.ShapeDtypeStruct((2, E, 128), jnp.float32),
            jax.ShapeDtypeStruct((2, E, 128), jnp.float32),
            jax.ShapeDtypeStruct((2, E, 128), jnp.float32),
        ],
    )(edge_attr, w_all, b_all)


def _node_call(x, agg, w1, b1, w2, b2):
    d = x.shape[1]
    nc, _, f = agg.shape
    return pl.pallas_call(
        _node_body,
        grid=(N // NB,),
        in_specs=[
            pl.BlockSpec((NB, d), lambda i: (i, 0)),
            pl.BlockSpec((nc, NB, f), lambda i: (0, i, 0)),
            pl.BlockSpec((d, HID), lambda i: (0, 0)),
            pl.BlockSpec((1, HID), lambda i: (0, 0)),
            pl.BlockSpec((HID, HID), lambda i: (0, 0)),
            pl.BlockSpec((1, HID), lambda i: (0, 0)),
        ],
        out_specs=[
            pl.BlockSpec((NB, HID), lambda i: (i, 0)),
            pl.BlockSpec((8, HID), lambda i: (0, 0)),
        ],
        out_shape=[
            jax.ShapeDtypeStruct((N, HID), jnp.float32),
            jax.ShapeDtypeStruct((8, HID), jnp.float32),
        ],
    )(x, agg, w1, b1, w2, b2)


def _bn_call(h2, st, gamma, beta):
    return pl.pallas_call(
        _bn_body,
        grid=(N // NB,),
        in_specs=[
            pl.BlockSpec((NB, HID), lambda i: (i, 0)),
            pl.BlockSpec((8, HID), lambda i: (0, 0)),
            pl.BlockSpec((1, HID), lambda i: (0, 0)),
            pl.BlockSpec((1, HID), lambda i: (0, 0)),
        ],
        out_specs=[
            pl.BlockSpec((NB, HID), lambda i: (i, 0)),
            pl.BlockSpec((2, NB, 128), lambda i: (0, i, 0)),
            pl.BlockSpec((8, HID), lambda i: (0, 0)),
        ],
        out_shape=[
            jax.ShapeDtypeStruct((N, HID), jnp.float32),
            jax.ShapeDtypeStruct((2, N, 128), jnp.float32),
            jax.ShapeDtypeStruct((8, HID), jnp.float32),
        ],
    )(h2, st, gamma, beta)


def _head_call(xsum, wm1, bm1, wm2p, bm2p):
    return pl.pallas_call(
        _head_body,
        in_specs=[
            pl.BlockSpec((8, HID), lambda: (0, 0)),
            pl.BlockSpec((HID, HID), lambda: (0, 0)),
            pl.BlockSpec((1, HID), lambda: (0, 0)),
            pl.BlockSpec((HID, 128), lambda: (0, 0)),
            pl.BlockSpec((1, 128), lambda: (0, 0)),
        ],
        out_specs=pl.BlockSpec((1, 128), lambda: (0, 0)),
        out_shape=jax.ShapeDtypeStruct((1, 128), jnp.float32),
    )(xsum, wm1, bm1, wm2p, bm2p)


def _sc_edge_body(F, nc_active, x_hbm, e_hbm, src_hbm, dst_hbm, out_hbm,
                  srcb, dstb, xbufs, ebufs, mbufs, zbuf, agg,
                  sidx, sx, se, ss):
    c = lax.axis_index("c")
    s = lax.axis_index("s")

    @pl.when(c < nc_active)
    def _body():
        _sc_edge_tile(F, c, s, x_hbm, e_hbm, src_hbm, dst_hbm, out_hbm,
                      srcb, dstb, xbufs, ebufs, mbufs, zbuf, agg,
                      sidx, sx, se, ss)


def _sc_edge_tile(F, c, s, x_hbm, e_hbm, src_hbm, dst_hbm, out_hbm,
                  srcb, dstb, xbufs, ebufs, mbufs, zbuf, agg,
                  sidx, sx, se, ss):
    def zrow(r, carry):
        for j in range(F // 16):
            zbuf[r, pl.ds(j * 16, 16)] = jnp.zeros((16,), jnp.float32)
        return carry
    lax.fori_loop(0, ZROWS, zrow, 0)
    for r in range(RPT // ZROWS):
        pltpu.sync_copy(zbuf, agg.at[pl.ds(s * RPT + r * ZROWS, ZROWS)])
    plsc.subcore_barrier()

    sbase = (c * NSUB + s) * EPT
    dbase = s * EPT
    ebase = (c * NSUB + s) * EPT

    def start_idx(k, q):
        pltpu.async_copy(src_hbm.at[pl.ds(sbase + k * CHUNK, CHUNK)],
                         srcb[q], sidx[q])
        pltpu.async_copy(dst_hbm.at[pl.ds(dbase + k * CHUNK, CHUNK)],
                         dstb[q], sidx[q])

    def wait_idx(q):
        pltpu.make_async_copy(src_hbm.at[pl.ds(sbase, CHUNK)], srcb[q],
                              sidx[q]).wait()
        pltpu.make_async_copy(dst_hbm.at[pl.ds(dbase, CHUNK)], dstb[q],
                              sidx[q]).wait()

    def start_data(k, q, b):
        pltpu.async_copy(x_hbm.at[srcb[q]], xbufs[b], sx[b])
        pltpu.async_copy(e_hbm.at[pl.ds(ebase + k * CHUNK, CHUNK)],
                         ebufs[b], se[b])

    def wait_data(b):
        pltpu.make_async_copy(x_hbm.at[srcb[0]], xbufs[b], sx[b]).wait()
        pltpu.make_async_copy(e_hbm.at[pl.ds(ebase, CHUNK)], ebufs[b],
                              se[b]).wait()

    def compute(b):
        eb, xb, mb = ebufs[b], xbufs[b], mbufs[b]

        def row(i, carry):
            for j in range(F // 16):
                v = eb[i, pl.ds(j * 16, 16)] + xb[i, pl.ds(j * 16, 16)]
                mb[i, pl.ds(j * 16, 16)] = jnp.maximum(v, 0.0)
            return carry
        lax.fori_loop(0, CHUNK, row, 0)

    def scatter(q, b):
        pltpu.async_copy(mbufs[b], agg.at[dstb[q]], ss[q], add=True)

    def wait_scatter(q, b):
        pltpu.make_async_copy(mbufs[b], agg.at[dstb[q]], ss[q]).wait()

    start_idx(0, 0)
    start_idx(1, 1)
    start_idx(2, 2)
    wait_idx(0)
    start_data(0, 0, 0)

    def step(i, carry):
        for u in range(4):
            k = 4 * i + u
            q = u
            b = u % 2

            def _pf():
                wait_idx((u + 1) % 4)
                start_data(k + 1, (u + 1) % 4, b ^ 1)
            if u == 3:
                @pl.when(i < (NCH // 4) - 1)
                def _():
                    _pf()
            else:
                _pf()

            wait_data(b)
            compute(b)

            def _w():
                wait_scatter((u + 3) % 4, b ^ 1)
            if u == 0:
                @pl.when(i > 0)
                def _():
                    _w()
            else:
                _w()

            def _si():
                start_idx(k + 3, (u + 3) % 4)
            if u == 0:
                _si()
            else:
                @pl.when(i < (NCH // 4) - 1)
                def _():
                    _si()

            scatter(q, b)
        return carry

    lax.fori_loop(0, NCH // 4, step, 0)
    wait_scatter(3, 1)
    plsc.subcore_barrier()

    pltpu.sync_copy(agg.at[pl.ds(s * RPT, RPT)], out_hbm.at[c * NSUB + s])


def _sc_edge_call(F, nc_active, x_pk, e_pk, src2, dst):
    mesh = plsc.VectorSubcoreMesh(core_axis_name="c", subcore_axis_name="s",
                                  num_cores=NCORE, num_subcores=NSUB)
    kern = functools.partial(
        pl.kernel,
        mesh=mesh,
        out_type=jax.ShapeDtypeStruct((nc_active * NSUB, RPT, F), jnp.float32),
        scratch_types=[
            tuple(pltpu.VMEM((CHUNK,), jnp.int32) for _ in range(4)),
            tuple(pltpu.VMEM((CHUNK,), jnp.int32) for _ in range(4)),
            tuple(pltpu.VMEM((CHUNK, F), jnp.float32) for _ in range(2)),
            tuple(pltpu.VMEM((CHUNK, F), jnp.float32) for _ in range(2)),
            tuple(pltpu.VMEM((CHUNK, F), jnp.float32) for _ in range(2)),
            pltpu.VMEM((ZROWS, F), jnp.float32),
            pltpu.VMEM_SHARED((N, F), jnp.float32),
            tuple(pltpu.SemaphoreType.DMA for _ in range(4)),
            tuple(pltpu.SemaphoreType.DMA for _ in range(2)),
            tuple(pltpu.SemaphoreType.DMA for _ in range(2)),
            tuple(pltpu.SemaphoreType.DMA for _ in range(4)),
        ],
    )(functools.partial(_sc_edge_body, F, nc_active))
    return kern(x_pk, e_pk, src2, dst)


def kernel(atomic_number, other_feats, edge_index, edge_attr, params):
    p = params
    layers = p["layers"]

    emb_wide = jnp.zeros((104, D0), jnp.float32).at[:100, :64].set(p["emb"])
    feats_pad = jnp.zeros((N, D0), jnp.float32).at[:, 64:72].set(other_feats)
    ids = atomic_number.reshape(N, 1).astype(jnp.int32)

    we0 = jnp.zeros((EDGE_DIM, D0), jnp.float32).at[:, :72].set(layers[0]["We"])
    be0 = jnp.zeros((D0,), jnp.float32).at[:72].set(layers[0]["be"])
    w_all = jnp.concatenate(
        [we0, layers[1]["We"], layers[2]["We"], layers[3]["We"]], axis=1)
    b_all = jnp.concatenate(
        [be0, layers[1]["be"], layers[2]["be"], layers[3]["be"]]).reshape(1, 896)

    w1_0 = jnp.zeros((D0, HID), jnp.float32).at[:72, :].set(layers[0]["W1"])

    src = edge_index[0].astype(jnp.int32)
    dst = edge_index[1].astype(jnp.int32)
    src2 = jnp.concatenate([src, src + N])
    dst_r = dst

    wm2p = jnp.zeros((HID, 128), jnp.float32).at[:, 0:1].set(p["Wm2"])
    bm2p = jnp.zeros((1, 128), jnp.float32).at[0, 0].set(p["bm2"][0])

    x = _embed_call(ids, feats_pad, emb_wide)
    xpk = x

    e0, e1, e2, e3 = _edge_call(edge_attr, w_all, b_all)
    e_pks = [e0, e1.reshape(2 * E, 128),
             e2.reshape(2 * E, 128), e3.reshape(2 * E, 128)]

    xsum = None
    for li in range(4):
        l = layers[li]
        nc = 1 if li == 0 else 2
        agg = _sc_edge_call(128, nc, xpk, e_pks[li], src2, dst_r)
        agg = agg.reshape(nc, N, 128)
        w1 = w1_0 if li == 0 else l["W1"]
        h2, st = _node_call(x, agg, w1, l["b1"].reshape(1, HID),
                            l["W2"], l["b2"].reshape(1, HID))
        x, xpk, xsum = _bn_call(h2, st, l["gamma"].reshape(1, HID),
                                l["beta"].reshape(1, HID))
        xpk = xpk.reshape(2 * N, 128)

    res = _head_call(xsum, p["Wm1"], p["bm1"].reshape(1, HID), wm2p, bm2p)
    return res[0, 0:1]
